# Initial kernel scaffold; baseline (speedup 1.0000x reference)
import jax
import jax.numpy as jnp
from jax.experimental import pallas as pl


def kernel(edge_index, user_idx, num_nodes, fraud_label_i, W1, b1, W2, b2):
    raise NotImplementedError("write your pallas kernel here")


# speedup vs baseline: 69.9625x; 69.9625x over previous
import jax
import jax.numpy as jnp
from jax.experimental import pallas as pl

NUM_USERS = 1024
NUM_ITEMS = 4096
TOPK = 10


def _sums_body(cnt_ref, cu_ref, ci_ref):
    cnt = cnt_ref[...]
    cu_ref[...] = jnp.sum(cnt, axis=1)
    ci_ref[...] = jnp.sum(cnt, axis=0)


def _dense_body(cnt_ref, du_ref, di_ref, wi_ref, s_ref):
    cnt = cnt_ref[...]
    su = jnp.sqrt(du_ref[...])
    sdi = jnp.sqrt(di_ref[...])
    x = (su[:, None] * sdi[None, :]) * wi_ref[...][None, :]
    a = jnp.zeros_like(cnt)
    for t in range(8):
        a = a + jnp.where(cnt > t, x, 0.0)
    a = a + jnp.maximum(cnt - 8.0, 0.0) * x
    c0 = jax.lax.dot_general(a, a, (((1,), (1,)), ((), ())),
                             preferred_element_type=jnp.float32)
    c = (c0 * su[:, None]) * su[None, :]
    iota = jax.lax.broadcasted_iota(jnp.int32, (NUM_USERS, NUM_USERS), 1)
    s0 = jnp.zeros((NUM_USERS, NUM_USERS), dtype=jnp.float32)
    for _ in range(TOPK):
        m = jnp.max(c, axis=1, keepdims=True)
        first = jnp.min(jnp.where(c == m, iota, NUM_USERS), axis=1,
                        keepdims=True)
        onehot = iota == first
        s0 = s0 + jnp.where(onehot & (m > 0), m * 0.5, 0.0)
        c = jnp.where(onehot, -jnp.inf, c)
    s_ref[...] = s0 + s0.T


def _histogram(edge_index):
    src, dst = edge_index[0], edge_index[1]
    mask = (src < NUM_USERS) & (dst >= NUM_USERS)
    u = jnp.where(mask, src, 0)
    i = jnp.where(mask, dst - NUM_USERS, 0)
    return jnp.zeros((NUM_USERS, NUM_ITEMS), jnp.float32).at[u, i].add(
        mask.astype(jnp.float32))


def kernel(edge_index, user_idx, num_nodes, fraud_label_i, W1, b1, W2, b2):
    cnt = _histogram(edge_index)
    cnt_u, cnt_i = pl.pallas_call(
        _sums_body,
        out_shape=(
            jax.ShapeDtypeStruct((NUM_USERS,), jnp.float32),
            jax.ShapeDtypeStruct((NUM_ITEMS,), jnp.float32),
        ),
    )(cnt)
    d_u = jnp.log1p(cnt_u)
    d_i = jnp.log1p(cnt_i)
    x_i = jnp.stack([d_i, fraud_label_i], axis=-1)
    h = jax.nn.relu(x_i @ W1.T + b1)
    w_i = jax.nn.sigmoid(h @ W2.T + b2).squeeze(-1)
    s = pl.pallas_call(
        _dense_body,
        out_shape=jax.ShapeDtypeStruct((NUM_USERS, NUM_USERS), jnp.float32),
    )(cnt, d_u, d_i, w_i)
    return s, d_u
